# baseline (speedup 1.0000x reference)
import functools
import jax
import jax.numpy as jnp
from jax.experimental import pallas as pl
from jax.experimental.pallas import tpu as pltpu


def _copy3(x_ref, o_ref):
    o_ref[...] = x_ref[...]


def _mkcopy3(x3, bt, name):
    B = x3.shape[0]
    rest = x3.shape[1:]
    body = functools.partial(_copy3)
    body.__name__ = name
    return pl.pallas_call(
        body,
        out_shape=jax.ShapeDtypeStruct(x3.shape, x3.dtype),
        grid=(B // bt,),
        in_specs=[pl.BlockSpec((bt,) + rest, lambda b: (b,) + (0,) * len(rest))],
        out_specs=pl.BlockSpec((bt,) + rest, lambda b: (b,) + (0,) * len(rest)),
        compiler_params=pltpu.CompilerParams(
            dimension_semantics=("parallel",),
            vmem_limit_bytes=100 << 20,
        ),
    )(x3)


@jax.jit
def _probe(x, w1, w2):
    B, C, H, W = x.shape
    HW = H * W
    a = _mkcopy3(x.reshape(B, C, HW), 2, "v1_c3136")
    b = _mkcopy3(x.reshape(B, C // 2, 2 * HW), 2, "v2_c6272")
    c = _mkcopy3(x.reshape(B, 2 * HW, C // 2), 2, "v3_r6272xc128")
    d = _mkcopy3(x.reshape(B * C * HW // 128 // 32, 32, 128), 98, "v4_flat128")
    e = _mkcopy3(x.reshape(B * 8, C * HW // 8), 8, "v5_big2d")
    s = a.sum() + b.sum() + c.sum() + d.sum() + e.sum()
    return x * 0 + s / x.size


def kernel(x, w1, w2):
    return _probe(x, w1, w2)


# speedup vs baseline: 6.7251x; 6.7251x over previous
import functools

import jax
import jax.numpy as jnp
from jax import lax
from jax.experimental import pallas as pl
from jax.experimental.pallas import tpu as pltpu


def _se_fused_kernel(x_ref, w1t_ref, w2t_ref, o_ref, *, inv_hw):
    x = x_ref[...]
    pooled = jnp.sum(x, axis=2, dtype=jnp.float32) * inv_hw
    h = jnp.dot(pooled, w1t_ref[...], preferred_element_type=jnp.float32)
    h = jnp.maximum(h, 0.0)
    logits = jnp.dot(h, w2t_ref[...], preferred_element_type=jnp.float32)
    gate = pl.reciprocal(1.0 + jnp.exp(-logits), approx=True)
    o_ref[...] = x * gate[:, :, None]


@functools.partial(jax.jit, static_argnames=("bt",))
def _se_forward(x, w1t, w2t, bt):
    B, C, H, W = x.shape
    HW = H * W
    Cr = w1t.shape[1]
    x3 = x.reshape(B, C, HW)
    out3 = pl.pallas_call(
        functools.partial(_se_fused_kernel, inv_hw=1.0 / HW),
        out_shape=jax.ShapeDtypeStruct((B, C, HW), x.dtype),
        grid=(B // bt,),
        in_specs=[
            pl.BlockSpec((bt, C, HW), lambda b: (b, 0, 0)),
            pl.BlockSpec((C, Cr), lambda b: (0, 0)),
            pl.BlockSpec((Cr, C), lambda b: (0, 0)),
        ],
        out_specs=pl.BlockSpec((bt, C, HW), lambda b: (b, 0, 0)),
        compiler_params=pltpu.CompilerParams(
            dimension_semantics=("parallel",),
            vmem_limit_bytes=100 << 20,
        ),
    )(x3, w1t, w2t)
    return out3.reshape(B, C, H, W)


def kernel(x, w1, w2):
    return _se_forward(x, w1.T, w2.T, bt=2)
